# baseline (speedup 1.0000x reference)
import functools

import jax
import jax.numpy as jnp
from jax import lax
from jax.experimental import pallas as pl
from jax.experimental.pallas import tpu as pltpu
from jax.experimental.pallas import tpu_sc as plsc

N_NODE = 1000000
EMD = 64
B = 16384
LABEL_SMOOTH = 0.1
LAMBDA_GEN = 1e-05

_NC = 2
_NS = 16
_NW = _NC * _NS
_BPW = B // _NW
_CHUNK = 128
_NCH = _BPW // _CHUNK

_BLK = 2048
_NBLK = B // _BLK


def _gather_body(ids_hbm, table_hbm, out_hbm, idx_v, rows_v, sem):
    wid = lax.axis_index("s") * _NC + lax.axis_index("c")
    base = wid * _BPW
    pltpu.sync_copy(ids_hbm.at[wid], idx_v)
    cps = [
        pltpu.async_copy(
            table_hbm.at[idx_v.at[j]],
            rows_v.at[pl.ds(j * _CHUNK, _CHUNK)],
            sem,
        )
        for j in range(_NCH)
    ]
    for cp in cps:
        cp.wait()
    pltpu.sync_copy(rows_v, out_hbm.at[pl.ds(base, _BPW)])


@functools.cache
def _sc_gather_fn():
    return functools.partial(
        pl.kernel,
        out_type=jax.ShapeDtypeStruct((B, EMD), jnp.float32),
        mesh=plsc.VectorSubcoreMesh(
            core_axis_name="c", subcore_axis_name="s", num_cores=_NC),
        scratch_types=[
            pltpu.VMEM((_NCH, _CHUNK), jnp.int32),
            pltpu.VMEM((_BPW, EMD), jnp.float32),
            pltpu.SemaphoreType.DMA,
        ],
        compiler_params=pltpu.CompilerParams(use_tc_tiling_on_sc=False),
    )(_gather_body)


def _dense_body(emb_ref, noise_ref, dis_ref, w_ref, b_ref,
                fake0_ref, fake1_ref, sc_ref, acc_ref):
    k = pl.program_id(0)

    @pl.when(k == 0)
    def _init():
        acc_ref[0] = 0.0
        acc_ref[1] = 0.0
        acc_ref[2] = 0.0

    emb = emb_ref[...]
    partial_emb = jnp.sum(emb * emb)

    ce = []
    for i in range(2):
        inp = emb + noise_ref[i]
        fake = jnp.dot(inp, w_ref[i], preferred_element_type=jnp.float32)
        fake = fake + b_ref[i]
        fake = jnp.where(fake >= 0, fake, 0.2 * fake)
        if i == 0:
            fake0_ref[...] = fake
        else:
            fake1_ref[...] = fake
        score = jnp.sum(dis_ref[i] * fake, axis=1, keepdims=True)
        ce_el = (jnp.maximum(score, 0.0) - score * (1.0 - LABEL_SMOOTH)
                 + jnp.log(1.0 + jnp.exp(-jnp.abs(score))))
        ce.append(jnp.sum(ce_el))

    acc_ref[0] = acc_ref[0] + ce[0]
    acc_ref[1] = acc_ref[1] + ce[1]
    acc_ref[2] = acc_ref[2] + partial_emb

    @pl.when(k == _NBLK - 1)
    def _fin():
        semb = acc_ref[2]
        w0 = w_ref[0]
        w1 = w_ref[1]
        n0 = (acc_ref[0] / B
              + LAMBDA_GEN * (0.5 * semb + 0.5 * jnp.sum(w0 * w0)))
        n1 = (acc_ref[1] / B
              + LAMBDA_GEN * (0.5 * semb + 0.5 * jnp.sum(w1 * w1)))
        sc_ref[0] = n0 + n1
        sc_ref[1] = n0
        sc_ref[2] = n1


def _dense(node_emb, noise, dis, w, b3, interpret=False):
    return pl.pallas_call(
        _dense_body,
        grid=(_NBLK,),
        in_specs=[
            pl.BlockSpec((_BLK, EMD), lambda k: (k, 0)),
            pl.BlockSpec((2, _BLK, EMD), lambda k: (0, k, 0)),
            pl.BlockSpec((2, _BLK, EMD), lambda k: (0, k, 0)),
            pl.BlockSpec((2, EMD, EMD), lambda k: (0, 0, 0)),
            pl.BlockSpec((2, 1, EMD), lambda k: (0, 0, 0)),
        ],
        out_specs=[
            pl.BlockSpec((_BLK, EMD), lambda k: (k, 0)),
            pl.BlockSpec((_BLK, EMD), lambda k: (k, 0)),
            pl.BlockSpec(memory_space=pltpu.SMEM),
        ],
        out_shape=[
            jax.ShapeDtypeStruct((B, EMD), jnp.float32),
            jax.ShapeDtypeStruct((B, EMD), jnp.float32),
            jax.ShapeDtypeStruct((3,), jnp.float32),
        ],
        scratch_shapes=[pltpu.SMEM((3,), jnp.float32)],
        compiler_params=pltpu.CompilerParams(
            dimension_semantics=("arbitrary",),
        ),
        interpret=interpret,
    )(node_emb, noise, dis, w, b3)


def kernel(node_ids, noise_embedding, dis_node_embedding, table, gen_w_1,
           gen_b_1):
    ids = node_ids.astype(jnp.int32).reshape(_NW, _NCH, _CHUNK)
    node_emb = _sc_gather_fn()(ids, table)
    b3 = gen_b_1.reshape(2, 1, EMD)
    fake0, fake1, sc = _dense(node_emb, noise_embedding,
                              dis_node_embedding, gen_w_1, b3)
    return (sc[0], fake0, fake1, sc[1], sc[2])
